# baseline (speedup 1.0000x reference)
import jax
import jax.numpy as jnp
from jax import lax
from jax.experimental import pallas as pl
from jax.experimental.pallas import tpu as pltpu
from jax.experimental.pallas import tpu_sc as plsc

_NUM_TASKS = 1000
_TOTAL_HEADS = 32
_NUM_HEADS = 8
_GROUPS = _TOTAL_HEADS // _NUM_HEADS
_TEMP = 5.0
_BATCH = 16384

_NC = 2
_NS = 16
_LANES = 16

_ROWS_PER_TILE = 64
_LAST_BASE = _NUM_TASKS - _ROWS_PER_TILE
_TBL = _NUM_TASKS * _NUM_HEADS
_B_PER_W = _BATCH // (_NC * _NS)
_OUT_W = _B_PER_W * _NUM_HEADS


def _body(hl_hbm, g1_hbm, g2_hbm, tids_hbm, outw_hbm, outi_hbm,
          hl_v, g1_v, g2_v, sw_v, tw_loc, ti_loc, tw_sh, ti_sh, tw_v,
          ti_v, tids_v, ow_v, oi_v):
    s = lax.axis_index("s")
    c = lax.axis_index("c")
    lane = lax.iota(jnp.int32, _LANES)
    lo8 = lane < _NUM_HEADS
    j = lane & (_NUM_HEADS - 1)

    rbase = jnp.minimum(s * _ROWS_PER_TILE, _LAST_BASE)
    pltpu.sync_copy(hl_hbm.at[pl.ds(rbase, _ROWS_PER_TILE)], hl_v)
    pltpu.sync_copy(g1_hbm.at[pl.ds(rbase, _ROWS_PER_TILE)], g1_v)
    pltpu.sync_copy(g2_hbm.at[pl.ds(rbase, _ROWS_PER_TILE)], g2_v)

    hi1 = jnp.where(lo8, 0, 1)

    def swap_halves(v):
        sw_v[pl.ds(0, _LANES)] = v
        sw_v[pl.ds(_LANES, _LANES)] = v
        return sw_v[pl.ds(_NUM_HEADS, _LANES)]

    def row_pair_best(r):
        a = (hl_v[r, pl.ds(0, _LANES)] + g1_v[r, pl.ds(0, _LANES)]
             - g2_v[r, pl.ds(0, _LANES)])
        b = (hl_v[r, pl.ds(_LANES, _LANES)] + g1_v[r, pl.ds(_LANES, _LANES)]
             - g2_v[r, pl.ds(_LANES, _LANES)])
        m = b > a
        return jnp.where(m, b, a), jnp.where(m, 2, 0) + hi1

    def chunk(ck, _):
        v0, g0 = row_pair_best(2 * ck)
        v1, g1 = row_pair_best(2 * ck + 1)
        cv = jnp.where(lo8, v0, v1)
        cg = jnp.where(lo8, g0, g1)
        fv = swap_halves(jnp.where(lo8, v1, v0))
        fg = swap_halves(jnp.where(lo8, g1, g0).astype(jnp.float32)).astype(jnp.int32)
        pick = (fv > cv) | ((fv == cv) & (fg < cg))
        bv = jnp.where(pick, fv, cv)
        bg = jnp.where(pick, fg, cg)
        sw = 1.0 / (1.0 + jnp.exp(-(bv / _TEMP)))
        tw_loc[pl.ds(ck * _LANES, _LANES)] = (1.0 - sw) + sw
        ti_loc[pl.ds(ck * _LANES, _LANES)] = bg * _NUM_HEADS + j
        return _

    lax.fori_loop(0, _ROWS_PER_TILE // 2, chunk, None)

    pltpu.sync_copy(tw_loc, tw_sh.at[pl.ds(rbase * _NUM_HEADS, _ROWS_PER_TILE * _NUM_HEADS)])
    pltpu.sync_copy(ti_loc, ti_sh.at[pl.ds(rbase * _NUM_HEADS, _ROWS_PER_TILE * _NUM_HEADS)])
    plsc.subcore_barrier()
    pltpu.sync_copy(tw_sh, tw_v)
    pltpu.sync_copy(ti_sh, ti_v)

    wid = c * _NS + s
    base = wid * _B_PER_W
    pltpu.sync_copy(tids_hbm.at[pl.ds(base, _B_PER_W)], tids_v)

    def pair(p, _):
        ridx = hi1 + 2 * p
        tpair = plsc.load_gather(tids_v, [ridx])
        widx = tpair * _NUM_HEADS + j
        plsc.store_scatter(ow_v, [ridx, j], plsc.load_gather(tw_v, [widx]))
        plsc.store_scatter(oi_v, [ridx, j], plsc.load_gather(ti_v, [widx]))
        return _

    lax.fori_loop(0, _B_PER_W // 2, pair, None)

    pltpu.sync_copy(ow_v, outw_hbm.at[pl.ds(base, _B_PER_W)])
    pltpu.sync_copy(oi_v, outi_hbm.at[pl.ds(base, _B_PER_W)])


_sc_call = pl.kernel(
    _body,
    out_type=(
        jax.ShapeDtypeStruct((_BATCH, _NUM_HEADS), jnp.float32),
        jax.ShapeDtypeStruct((_BATCH, _NUM_HEADS), jnp.int32),
    ),
    mesh=plsc.VectorSubcoreMesh(core_axis_name="c", subcore_axis_name="s"),
    scratch_types=[
        pltpu.VMEM((_ROWS_PER_TILE, _TOTAL_HEADS), jnp.float32),
        pltpu.VMEM((_ROWS_PER_TILE, _TOTAL_HEADS), jnp.float32),
        pltpu.VMEM((_ROWS_PER_TILE, _TOTAL_HEADS), jnp.float32),
        pltpu.VMEM((2 * _LANES,), jnp.float32),
        pltpu.VMEM((_ROWS_PER_TILE * _NUM_HEADS,), jnp.float32),
        pltpu.VMEM((_ROWS_PER_TILE * _NUM_HEADS,), jnp.int32),
        pltpu.VMEM_SHARED((_TBL,), jnp.float32),
        pltpu.VMEM_SHARED((_TBL,), jnp.int32),
        pltpu.VMEM((_TBL,), jnp.float32),
        pltpu.VMEM((_TBL,), jnp.int32),
        pltpu.VMEM((_B_PER_W,), jnp.int32),
        pltpu.VMEM((_B_PER_W, _NUM_HEADS), jnp.float32),
        pltpu.VMEM((_B_PER_W, _NUM_HEADS), jnp.int32),
    ],
    compiler_params=pltpu.CompilerParams(needs_layout_passes=False,
                                         use_tc_tiling_on_sc=False),
)


def kernel(task_ids, layer_idx, head_logits, gumbels1, gumbels2):
    hl = lax.dynamic_index_in_dim(head_logits, layer_idx, 1, keepdims=False)
    g1 = lax.dynamic_index_in_dim(gumbels1, layer_idx, 1, keepdims=False)
    g2 = lax.dynamic_index_in_dim(gumbels2, layer_idx, 1, keepdims=False)
    outw, outi = _sc_call(hl, g1, g2, task_ids.astype(jnp.int32))
    return (outi, outw)


# speedup vs baseline: 1.0597x; 1.0597x over previous
import jax
import jax.numpy as jnp
from jax import lax
from jax.experimental import pallas as pl
from jax.experimental.pallas import tpu as pltpu
from jax.experimental.pallas import tpu_sc as plsc

_NUM_TASKS = 1000
_TOTAL_HEADS = 32
_NUM_HEADS = 8
_GROUPS = _TOTAL_HEADS // _NUM_HEADS
_BATCH = 16384

_NC = 2
_NS = 16
_LANES = 16

_ROWS_PER_TILE = 64
_LAST_BASE = _NUM_TASKS - _ROWS_PER_TILE
_TBL = _NUM_TASKS * _NUM_HEADS
_B_PER_W = _BATCH // (_NC * _NS)
_OUT_W = _B_PER_W * _NUM_HEADS


def _body(sc_hbm, tids_hbm, outw_hbm, outi_hbm,
          hl_v, g1_v, g2_v, sw_v, ti_loc, ti_sh, ti_v,
          tids_v, ow_v, oi_v):
    s = lax.axis_index("s")
    c = lax.axis_index("c")
    lane = lax.iota(jnp.int32, _LANES)
    lo8 = lane < _NUM_HEADS
    j = lane & (_NUM_HEADS - 1)
    ones16 = jnp.full((_LANES,), 1.0, jnp.float32)

    rbase = jnp.minimum(s * _ROWS_PER_TILE, _LAST_BASE)
    pltpu.sync_copy(sc_hbm.at[0, pl.ds(rbase, _ROWS_PER_TILE)], hl_v)
    pltpu.sync_copy(sc_hbm.at[1, pl.ds(rbase, _ROWS_PER_TILE)], g1_v)
    pltpu.sync_copy(sc_hbm.at[2, pl.ds(rbase, _ROWS_PER_TILE)], g2_v)

    hi1 = jnp.where(lo8, 0, 1)

    def swap_halves(v):
        sw_v[pl.ds(0, _LANES)] = v
        sw_v[pl.ds(_LANES, _LANES)] = v
        return sw_v[pl.ds(_NUM_HEADS, _LANES)]

    def row_pair_best(r):
        a = (hl_v[r, pl.ds(0, _LANES)] + g1_v[r, pl.ds(0, _LANES)]
             - g2_v[r, pl.ds(0, _LANES)])
        b = (hl_v[r, pl.ds(_LANES, _LANES)] + g1_v[r, pl.ds(_LANES, _LANES)]
             - g2_v[r, pl.ds(_LANES, _LANES)])
        m = b > a
        return jnp.where(m, b, a), jnp.where(m, 2, 0) + hi1

    def chunk(ck, _):
        v0, g0 = row_pair_best(2 * ck)
        v1, g1 = row_pair_best(2 * ck + 1)
        cv = jnp.where(lo8, v0, v1)
        cg = jnp.where(lo8, g0, g1)
        fv = swap_halves(jnp.where(lo8, v1, v0))
        fg = swap_halves(jnp.where(lo8, g1, g0).astype(jnp.float32)).astype(jnp.int32)
        pick = (fv > cv) | ((fv == cv) & (fg < cg))
        bg = jnp.where(pick, fg, cg)
        ti_loc[pl.ds(ck * _LANES, _LANES)] = bg * _NUM_HEADS + j
        return _

    lax.fori_loop(0, _ROWS_PER_TILE // 2, chunk, None)

    pltpu.sync_copy(ti_loc, ti_sh.at[pl.ds(rbase * _NUM_HEADS, _ROWS_PER_TILE * _NUM_HEADS)])
    plsc.subcore_barrier()
    pltpu.sync_copy(ti_sh, ti_v)

    wid = c * _NS + s
    base = wid * _B_PER_W
    pltpu.sync_copy(tids_hbm.at[pl.ds(base, _B_PER_W)], tids_v)

    def pair(p, _):
        ridx = hi1 + 2 * p
        tpair = plsc.load_gather(tids_v, [ridx])
        widx = tpair * _NUM_HEADS + j
        plsc.store_scatter(oi_v, [ridx, j], plsc.load_gather(ti_v, [widx]))
        plsc.store_scatter(ow_v, [ridx, j], ones16)
        return _

    lax.fori_loop(0, _B_PER_W // 2, pair, None)

    pltpu.sync_copy(ow_v, outw_hbm.at[pl.ds(base, _B_PER_W)])
    pltpu.sync_copy(oi_v, outi_hbm.at[pl.ds(base, _B_PER_W)])


_sc_call = pl.kernel(
    _body,
    out_type=(
        jax.ShapeDtypeStruct((_BATCH, _NUM_HEADS), jnp.float32),
        jax.ShapeDtypeStruct((_BATCH, _NUM_HEADS), jnp.int32),
    ),
    mesh=plsc.VectorSubcoreMesh(core_axis_name="c", subcore_axis_name="s"),
    scratch_types=[
        pltpu.VMEM((_ROWS_PER_TILE, _TOTAL_HEADS), jnp.float32),
        pltpu.VMEM((_ROWS_PER_TILE, _TOTAL_HEADS), jnp.float32),
        pltpu.VMEM((_ROWS_PER_TILE, _TOTAL_HEADS), jnp.float32),
        pltpu.VMEM((2 * _LANES,), jnp.float32),
        pltpu.VMEM((_ROWS_PER_TILE * _NUM_HEADS,), jnp.int32),
        pltpu.VMEM_SHARED((_TBL,), jnp.int32),
        pltpu.VMEM((_TBL,), jnp.int32),
        pltpu.VMEM((_B_PER_W,), jnp.int32),
        pltpu.VMEM((_B_PER_W, _NUM_HEADS), jnp.float32),
        pltpu.VMEM((_B_PER_W, _NUM_HEADS), jnp.int32),
    ],
    compiler_params=pltpu.CompilerParams(needs_layout_passes=False,
                                         use_tc_tiling_on_sc=False),
)


def kernel(task_ids, layer_idx, head_logits, gumbels1, gumbels2):
    hl = lax.dynamic_index_in_dim(head_logits, layer_idx, 1, keepdims=False)
    g1 = lax.dynamic_index_in_dim(gumbels1, layer_idx, 1, keepdims=False)
    g2 = lax.dynamic_index_in_dim(gumbels2, layer_idx, 1, keepdims=False)
    stacked = jnp.stack([hl, g1, g2])
    outw, outi = _sc_call(stacked, task_ids.astype(jnp.int32))
    return (outi, outw)


# speedup vs baseline: 2.1128x; 1.9937x over previous
import jax
import jax.numpy as jnp
from jax import lax
from jax.experimental import pallas as pl
from jax.experimental.pallas import tpu as pltpu
from jax.experimental.pallas import tpu_sc as plsc

_NUM_TASKS = 1000
_TOTAL_HEADS = 32
_NUM_HEADS = 8
_GROUPS = _TOTAL_HEADS // _NUM_HEADS
_BATCH = 16384

_NC = 2
_NS = 16
_LANES = 16

_ROWS_PER_TILE = 64
_LAST_BASE = _NUM_TASKS - _ROWS_PER_TILE
_TBL = _NUM_TASKS * _NUM_HEADS
_B_PER_W = _BATCH // (_NC * _NS)
_OUT_W = _B_PER_W * _NUM_HEADS
_BLK = 128
_BLOCKS = _BATCH // _BLK
_BLOCKS_PER_W = _B_PER_W // _BLK


def _body(sc_hbm, tids_hbm, outw_hbm, outi_hbm,
          hl_v, g1_v, g2_v, sw_v, ti_loc, ti_sh, ti_v,
          tids_v, ow_v, oi_v):
    s = lax.axis_index("s")
    c = lax.axis_index("c")
    lane = lax.iota(jnp.int32, _LANES)
    lo8 = lane < _NUM_HEADS
    j = lane & (_NUM_HEADS - 1)
    ones16 = jnp.full((_LANES,), 1.0, jnp.float32)

    rbase = jnp.minimum(s * _ROWS_PER_TILE, _LAST_BASE)
    pltpu.sync_copy(sc_hbm.at[0, pl.ds(rbase, _ROWS_PER_TILE)], hl_v)
    pltpu.sync_copy(sc_hbm.at[1, pl.ds(rbase, _ROWS_PER_TILE)], g1_v)
    pltpu.sync_copy(sc_hbm.at[2, pl.ds(rbase, _ROWS_PER_TILE)], g2_v)

    hi1 = jnp.where(lo8, 0, 1)

    def swap_halves(v):
        sw_v[pl.ds(0, _LANES)] = v
        sw_v[pl.ds(_LANES, _LANES)] = v
        return sw_v[pl.ds(_NUM_HEADS, _LANES)]

    def row_pair_best(r):
        a = (hl_v[r, pl.ds(0, _LANES)] + g1_v[r, pl.ds(0, _LANES)]
             - g2_v[r, pl.ds(0, _LANES)])
        b = (hl_v[r, pl.ds(_LANES, _LANES)] + g1_v[r, pl.ds(_LANES, _LANES)]
             - g2_v[r, pl.ds(_LANES, _LANES)])
        m = b > a
        return jnp.where(m, b, a), jnp.where(m, 2, 0) + hi1

    def chunk(ck, _):
        v0, g0 = row_pair_best(2 * ck)
        v1, g1 = row_pair_best(2 * ck + 1)
        cv = jnp.where(lo8, v0, v1)
        cg = jnp.where(lo8, g0, g1)
        fv = swap_halves(jnp.where(lo8, v1, v0))
        fg = swap_halves(jnp.where(lo8, g1, g0).astype(jnp.float32)).astype(jnp.int32)
        pick = (fv > cv) | ((fv == cv) & (fg < cg))
        bg = jnp.where(pick, fg, cg)
        ti_loc[pl.ds(ck * _LANES, _LANES)] = bg * _NUM_HEADS + j
        return _

    lax.fori_loop(0, _ROWS_PER_TILE // 2, chunk, None)

    pltpu.sync_copy(ti_loc, ti_sh.at[pl.ds(rbase * _NUM_HEADS, _ROWS_PER_TILE * _NUM_HEADS)])
    plsc.subcore_barrier()
    pltpu.sync_copy(ti_sh, ti_v)

    wid = c * _NS + s
    base = wid * _B_PER_W
    pltpu.sync_copy(tids_hbm.at[pl.ds(base, _B_PER_W)], tids_v)

    def grp(q, _):
        tv8 = tids_v[pl.ds(q * _LANES, _LANES)] * _NUM_HEADS
        blk = q >> 3
        off = (q & 7) * _LANES
        for h in range(_NUM_HEADS):
            oi_v[blk, h, pl.ds(off, _LANES)] = plsc.load_gather(ti_v, [tv8 + h])
            ow_v[blk, h, pl.ds(off, _LANES)] = ones16
        return _

    lax.fori_loop(0, _B_PER_W // _LANES, grp, None)

    pltpu.sync_copy(ow_v, outw_hbm.at[pl.ds(wid * _BLOCKS_PER_W, _BLOCKS_PER_W)])
    pltpu.sync_copy(oi_v, outi_hbm.at[pl.ds(wid * _BLOCKS_PER_W, _BLOCKS_PER_W)])


_sc_call = pl.kernel(
    _body,
    out_type=(
        jax.ShapeDtypeStruct((_BLOCKS, _NUM_HEADS, _BLK), jnp.float32),
        jax.ShapeDtypeStruct((_BLOCKS, _NUM_HEADS, _BLK), jnp.int32),
    ),
    mesh=plsc.VectorSubcoreMesh(core_axis_name="c", subcore_axis_name="s"),
    scratch_types=[
        pltpu.VMEM((_ROWS_PER_TILE, _TOTAL_HEADS), jnp.float32),
        pltpu.VMEM((_ROWS_PER_TILE, _TOTAL_HEADS), jnp.float32),
        pltpu.VMEM((_ROWS_PER_TILE, _TOTAL_HEADS), jnp.float32),
        pltpu.VMEM((2 * _LANES,), jnp.float32),
        pltpu.VMEM((_ROWS_PER_TILE * _NUM_HEADS,), jnp.int32),
        pltpu.VMEM_SHARED((_TBL,), jnp.int32),
        pltpu.VMEM((_TBL,), jnp.int32),
        pltpu.VMEM((_B_PER_W,), jnp.int32),
        pltpu.VMEM((_BLOCKS_PER_W, _NUM_HEADS, _BLK), jnp.float32),
        pltpu.VMEM((_BLOCKS_PER_W, _NUM_HEADS, _BLK), jnp.int32),
    ],
    compiler_params=pltpu.CompilerParams(needs_layout_passes=False,
                                         use_tc_tiling_on_sc=False),
)


def kernel(task_ids, layer_idx, head_logits, gumbels1, gumbels2):
    hl = lax.dynamic_index_in_dim(head_logits, layer_idx, 1, keepdims=False)
    g1 = lax.dynamic_index_in_dim(gumbels1, layer_idx, 1, keepdims=False)
    g2 = lax.dynamic_index_in_dim(gumbels2, layer_idx, 1, keepdims=False)
    stacked = jnp.stack([hl, g1, g2])
    outw, outi = _sc_call(stacked, task_ids.astype(jnp.int32))
    outw = outw.transpose(0, 2, 1).reshape(_BATCH, _NUM_HEADS)
    outi = outi.transpose(0, 2, 1).reshape(_BATCH, _NUM_HEADS)
    return (outi, outw)


# speedup vs baseline: 2.2582x; 1.0688x over previous
import jax
import jax.numpy as jnp
from jax import lax
from jax.experimental import pallas as pl
from jax.experimental.pallas import tpu as pltpu
from jax.experimental.pallas import tpu_sc as plsc

_NUM_TASKS = 1000
_TOTAL_HEADS = 32
_NUM_HEADS = 8
_GROUPS = _TOTAL_HEADS // _NUM_HEADS
_BATCH = 16384

_NC = 2
_NS = 16
_LANES = 16

_ROWS_PER_TILE = 64
_LAST_BASE = _NUM_TASKS - _ROWS_PER_TILE
_TBL = _NUM_TASKS * _NUM_HEADS
_B_PER_W = _BATCH // (_NC * _NS)
_OUT_W = _B_PER_W * _NUM_HEADS
_BLK = 128
_BLOCKS = _BATCH // _BLK
_BLOCKS_PER_W = _B_PER_W // _BLK


def _body(sc_hbm, tids_hbm, outw_hbm, outi_hbm,
          hl_v, g1_v, g2_v, sw_v, ti_loc, ti_sh, ti_v,
          tids_v, ow_v, oi_v, dsem):
    s = lax.axis_index("s")
    c = lax.axis_index("c")
    lane = lax.iota(jnp.int32, _LANES)
    lo8 = lane < _NUM_HEADS
    j = lane & (_NUM_HEADS - 1)
    ones16 = jnp.full((_LANES,), 1.0, jnp.float32)

    wid = c * _NS + s
    base = wid * _B_PER_W
    rbase = jnp.minimum(s * _ROWS_PER_TILE, _LAST_BASE)
    cp_hl = pltpu.async_copy(sc_hbm.at[0, pl.ds(rbase, _ROWS_PER_TILE)], hl_v, dsem)
    cp_g1 = pltpu.async_copy(sc_hbm.at[1, pl.ds(rbase, _ROWS_PER_TILE)], g1_v, dsem)
    cp_g2 = pltpu.async_copy(sc_hbm.at[2, pl.ds(rbase, _ROWS_PER_TILE)], g2_v, dsem)
    cp_ti = pltpu.async_copy(tids_hbm.at[pl.ds(base, _B_PER_W)], tids_v, dsem)
    cp_hl.wait()
    cp_g1.wait()
    cp_g2.wait()
    cp_ti.wait()

    hi1 = jnp.where(lo8, 0, 1)

    def swap_halves(v):
        sw_v[pl.ds(0, _LANES)] = v
        sw_v[pl.ds(_LANES, _LANES)] = v
        return sw_v[pl.ds(_NUM_HEADS, _LANES)]

    def row_pair_best(r):
        a = (hl_v[r, pl.ds(0, _LANES)] + g1_v[r, pl.ds(0, _LANES)]
             - g2_v[r, pl.ds(0, _LANES)])
        b = (hl_v[r, pl.ds(_LANES, _LANES)] + g1_v[r, pl.ds(_LANES, _LANES)]
             - g2_v[r, pl.ds(_LANES, _LANES)])
        m = b > a
        return jnp.where(m, b, a), jnp.where(m, 2, 0) + hi1

    def chunk(ck, _):
        v0, g0 = row_pair_best(2 * ck)
        v1, g1 = row_pair_best(2 * ck + 1)
        cv = jnp.where(lo8, v0, v1)
        cg = jnp.where(lo8, g0, g1)
        fv = swap_halves(jnp.where(lo8, v1, v0))
        fg = swap_halves(jnp.where(lo8, g1, g0).astype(jnp.float32)).astype(jnp.int32)
        pick = (fv > cv) | ((fv == cv) & (fg < cg))
        bg = jnp.where(pick, fg, cg)
        ti_loc[pl.ds(ck * _LANES, _LANES)] = bg * _NUM_HEADS + j
        return _

    lax.fori_loop(0, _ROWS_PER_TILE // 2, chunk, None)

    pltpu.sync_copy(ti_loc, ti_sh.at[pl.ds(rbase * _NUM_HEADS, _ROWS_PER_TILE * _NUM_HEADS)])
    plsc.subcore_barrier()
    pltpu.sync_copy(ti_sh, ti_v)

    def grp(q, _):
        tv8 = tids_v[pl.ds(q * _LANES, _LANES)] * _NUM_HEADS
        blk = q >> 3
        off = (q & 7) * _LANES
        for h in range(_NUM_HEADS):
            oi_v[blk, h, pl.ds(off, _LANES)] = plsc.load_gather(ti_v, [tv8 + h])
            ow_v[blk, h, pl.ds(off, _LANES)] = ones16
        return _

    lax.fori_loop(0, _B_PER_W // _LANES, grp, None)

    cp_ow = pltpu.async_copy(ow_v, outw_hbm.at[pl.ds(wid * _BLOCKS_PER_W, _BLOCKS_PER_W)], dsem)
    cp_oi = pltpu.async_copy(oi_v, outi_hbm.at[pl.ds(wid * _BLOCKS_PER_W, _BLOCKS_PER_W)], dsem)
    cp_ow.wait()
    cp_oi.wait()


_sc_call = pl.kernel(
    _body,
    out_type=(
        jax.ShapeDtypeStruct((_BLOCKS, _NUM_HEADS, _BLK), jnp.float32),
        jax.ShapeDtypeStruct((_BLOCKS, _NUM_HEADS, _BLK), jnp.int32),
    ),
    mesh=plsc.VectorSubcoreMesh(core_axis_name="c", subcore_axis_name="s"),
    scratch_types=[
        pltpu.VMEM((_ROWS_PER_TILE, _TOTAL_HEADS), jnp.float32),
        pltpu.VMEM((_ROWS_PER_TILE, _TOTAL_HEADS), jnp.float32),
        pltpu.VMEM((_ROWS_PER_TILE, _TOTAL_HEADS), jnp.float32),
        pltpu.VMEM((2 * _LANES,), jnp.float32),
        pltpu.VMEM((_ROWS_PER_TILE * _NUM_HEADS,), jnp.int32),
        pltpu.VMEM_SHARED((_TBL,), jnp.int32),
        pltpu.VMEM((_TBL,), jnp.int32),
        pltpu.VMEM((_B_PER_W,), jnp.int32),
        pltpu.VMEM((_BLOCKS_PER_W, _NUM_HEADS, _BLK), jnp.float32),
        pltpu.VMEM((_BLOCKS_PER_W, _NUM_HEADS, _BLK), jnp.int32),
        pltpu.SemaphoreType.DMA,
    ],
    compiler_params=pltpu.CompilerParams(needs_layout_passes=False,
                                         use_tc_tiling_on_sc=False),
)


def kernel(task_ids, layer_idx, head_logits, gumbels1, gumbels2):
    hl = lax.dynamic_index_in_dim(head_logits, layer_idx, 1, keepdims=False)
    g1 = lax.dynamic_index_in_dim(gumbels1, layer_idx, 1, keepdims=False)
    g2 = lax.dynamic_index_in_dim(gumbels2, layer_idx, 1, keepdims=False)
    stacked = jnp.stack([hl, g1, g2])
    outw, outi = _sc_call(stacked, task_ids.astype(jnp.int32))
    outw = outw.transpose(0, 2, 1).reshape(_BATCH, _NUM_HEADS)
    outi = outi.transpose(0, 2, 1).reshape(_BATCH, _NUM_HEADS)
    return (outi, outw)
